# baseline (speedup 1.0000x reference)
import functools

import jax
import jax.numpy as jnp
from jax import lax
from jax.experimental import pallas as pl
from jax.experimental.pallas import tpu as pltpu
from jax.experimental.pallas import tpu_sc as plsc

N = 10000
E = 320000
D = 128
R = 3

NTILES = 16
CHUNK = 128
GB = 4
EPT = 20480
E_PAD = NTILES * EPT
NBLK = E_PAD // CHUNK
BPT = EPT // CHUNK
IDXB = 40
NSTG = BPT // IDXB
TBL = 30080
ZPT = TBL // NTILES
CPT = 1872
QW = 32

_sc_mesh = plsc.VectorSubcoreMesh(core_axis_name="c", subcore_axis_name="s")


@functools.partial(
    pl.kernel,
    out_type=jax.ShapeDtypeStruct((4, R * N, QW), jnp.float32),
    mesh=_sc_mesh,
    scratch_types=[
        pltpu.VMEM((GB * CHUNK, QW), jnp.float32),
        pltpu.VMEM((GB * CHUNK, QW), jnp.float32),
        pltpu.VMEM((IDXB, CHUNK), jnp.int32),
        pltpu.VMEM((IDXB, CHUNK), jnp.int32),
        pltpu.VMEM_SHARED((TBL, QW), jnp.float32),
        pltpu.SemaphoreType.DMA,
        pltpu.SemaphoreType.DMA,
        pltpu.SemaphoreType.DMA,
    ],
    compiler_params=pltpu.CompilerParams(use_tc_tiling_on_sc=False),
)
def _sc_segsum(hq, srcs, keys, out, rows_v, rows_w, src_all, key_all, table,
               sem0, sem1, sem_s):
    c = lax.axis_index("c")
    s = lax.axis_index("s")

    zero = jnp.zeros((16,), jnp.float32)

    def _zrow(i, carry):
        rows_v[i, pl.ds(0, 16)] = zero
        rows_v[i, pl.ds(16, 16)] = zero
        return carry

    lax.fori_loop(0, GB * CHUNK, _zrow, 0)

    def _zero_table():
        z0 = s * ZPT
        pltpu.sync_copy(rows_v.at[pl.ds(0, 512)], table.at[pl.ds(z0, 512)])
        pltpu.sync_copy(rows_v.at[pl.ds(0, 512)], table.at[pl.ds(z0 + 512, 512)])
        pltpu.sync_copy(rows_v.at[pl.ds(0, 512)], table.at[pl.ds(z0 + 1024, 512)])
        pltpu.sync_copy(rows_v.at[pl.ds(0, ZPT - 1536)],
                        table.at[pl.ds(z0 + 1536, ZPT - 1536)])

    _zero_table()

    plsc.subcore_barrier()

    def _drain_body():
        for _ in range(2 * GB):
            pltpu.make_async_copy(rows_v.at[pl.ds(0, CHUNK)],
                                  table.at[key_all.at[0]], sem_s).wait()

    def _run_pass(gather_ref):
        for batch in range(NSTG):
            i0 = s * BPT + batch * IDXB
            pltpu.sync_copy(srcs.at[pl.ds(i0, IDXB)], src_all)
            pltpu.sync_copy(keys.at[pl.ds(i0, IDXB)], key_all)

            def _body(i, carry):
                @pl.when(jnp.logical_or(i != 0, batch != 0))
                def _():
                    _drain_body()

                b0 = i * (2 * GB)
                g0 = [
                    pltpu.async_copy(gather_ref.at[src_all.at[b0 + j]],
                                     rows_v.at[pl.ds(CHUNK * j, CHUNK)], sem0)
                    for j in range(GB)
                ]
                g1 = [
                    pltpu.async_copy(gather_ref.at[src_all.at[b0 + GB + j]],
                                     rows_w.at[pl.ds(CHUNK * j, CHUNK)], sem1)
                    for j in range(GB)
                ]
                for cp in g0:
                    cp.wait()
                for j in range(GB):
                    pltpu.async_copy(rows_v.at[pl.ds(CHUNK * j, CHUNK)],
                                     table.at[key_all.at[b0 + j]], sem_s,
                                     add=True)
                for cp in g1:
                    cp.wait()
                for j in range(GB):
                    pltpu.async_copy(rows_w.at[pl.ds(CHUNK * j, CHUNK)],
                                     table.at[key_all.at[b0 + GB + j]], sem_s,
                                     add=True)
                return carry

            lax.fori_loop(0, IDXB // (2 * GB), _body, 0)
        _drain_body()

    def _copy_out(q):
        o0 = s * CPT
        pltpu.sync_copy(table.at[pl.ds(o0, CPT)], out.at[q, pl.ds(o0, CPT)])

        @pl.when(s == NTILES - 1)
        def _tail():
            t0 = NTILES * CPT
            pltpu.sync_copy(table.at[pl.ds(t0, R * N - t0)],
                            out.at[q, pl.ds(t0, R * N - t0)])

    _run_pass(hq.at[2 * c])
    plsc.subcore_barrier()
    _copy_out(2 * c)
    plsc.subcore_barrier()
    lax.fori_loop(0, GB * CHUNK, _zrow, 0)
    _zero_table()
    plsc.subcore_barrier()

    _run_pass(hq.at[2 * c + 1])
    plsc.subcore_barrier()
    _copy_out(2 * c + 1)


BN = 1000


def _tc_self_body(h_ref, wself_ref, bias_ref, out_ref):
    out_ref[...] = (jnp.dot(h_ref[...], wself_ref[...],
                            preferred_element_type=jnp.float32)
                    + bias_ref[...])


def _tc_self(h, w_self, bias2):
    return pl.pallas_call(
        _tc_self_body,
        grid=(N // BN,),
        in_specs=[
            pl.BlockSpec((BN, D), lambda i: (i, 0)),
            pl.BlockSpec((D, D), lambda i: (0, 0)),
            pl.BlockSpec((1, D), lambda i: (0, 0)),
        ],
        out_specs=pl.BlockSpec((BN, D), lambda i: (i, 0)),
        out_shape=jax.ShapeDtypeStruct((N, D), jnp.float32),
    )(h, w_self, bias2)


def _tc_merge_body(s_ref, hw_ref, wc_ref, bases_ref, out_ref, outq_ref):
    acc = hw_ref[...]
    for r in range(R):
        w_r = (wc_ref[r, 0] * bases_ref[0]
               + wc_ref[r, 1] * bases_ref[1]
               + wc_ref[r, 2] * bases_ref[2])
        s_cat = jnp.concatenate([s_ref[q, r] for q in range(4)], axis=1)
        acc += jnp.dot(s_cat, w_r, preferred_element_type=jnp.float32)
    out_ref[...] = acc
    if outq_ref is not None:
        for q in range(4):
            outq_ref[q] = acc[:, QW * q:QW * (q + 1)]


def _tc_merge(s4, hw, w_comp, bases, with_hq):
    body = _tc_merge_body if with_hq else (
        lambda s_ref, hw_ref, wc_ref, bases_ref, out_ref:
        _tc_merge_body(s_ref, hw_ref, wc_ref, bases_ref, out_ref, None))
    out_shapes = [jax.ShapeDtypeStruct((N, D), jnp.float32)]
    out_specs = [pl.BlockSpec((BN, D), lambda i: (i, 0))]
    if with_hq:
        out_shapes.append(jax.ShapeDtypeStruct((4, N, QW), jnp.float32))
        out_specs.append(pl.BlockSpec((4, BN, QW), lambda i: (0, i, 0)))
    return pl.pallas_call(
        body,
        grid=(N // BN,),
        in_specs=[
            pl.BlockSpec((4, R, BN, QW), lambda i: (0, 0, i, 0)),
            pl.BlockSpec((BN, D), lambda i: (i, 0)),
            pl.BlockSpec(memory_space=pltpu.SMEM),
            pl.BlockSpec((R, D, D), lambda i: (0, 0, 0)),
        ],
        out_specs=out_specs,
        out_shape=out_shapes,
    )(s4, hw, w_comp, bases)


def kernel(features, edge_index, etypes,
           w_comp0, bases0, w_self0, bias0,
           w_comp1, bases1, w_self1, bias1,
           w_comp2, bases2, w_self2, bias2):
    src = edge_index[0]
    dst = edge_index[1]
    pad = E_PAD - E

    pad_src = jnp.arange(pad, dtype=jnp.int32) % N
    srcs = jnp.concatenate([src, pad_src]).reshape(NBLK, CHUNK)

    keys = etypes * N + dst
    pad_keys = R * N + (jnp.arange(pad, dtype=jnp.int32) % 16)
    keys_b = jnp.concatenate([keys, pad_keys]).reshape(NBLK, CHUNK)

    h = features
    hq = features.reshape(N, 4, QW).transpose(1, 0, 2)
    layers = ((w_comp0, bases0, w_self0, bias0),
              (w_comp1, bases1, w_self1, bias1),
              (w_comp2, bases2, w_self2, bias2))
    for li, (wc, ba, ws, bi) in enumerate(layers):
        s_acc = _sc_segsum(hq, srcs, keys_b)
        hw = _tc_self(h, ws, bi.reshape(1, D))
        s4 = s_acc.reshape(4, R, N, QW)
        if li < 2:
            h, hq = _tc_merge(s4, hw, wc, ba, True)
        else:
            (h,) = _tc_merge(s4, hw, wc, ba, False)
    return h


# speedup vs baseline: 1.0916x; 1.0916x over previous
import functools

import jax
import jax.numpy as jnp
from jax import lax
from jax.experimental import pallas as pl
from jax.experimental.pallas import tpu as pltpu
from jax.experimental.pallas import tpu_sc as plsc

N = 10000
E = 320000
D = 128
R = 3

NTILES = 16
CHUNK = 128
GB = 5
EPT = 20480
E_PAD = NTILES * EPT
NBLK = E_PAD // CHUNK
BPT = EPT // CHUNK
TBL = 30080
ZPT = TBL // NTILES
CPT = 1872
QW = 32

_sc_mesh = plsc.VectorSubcoreMesh(core_axis_name="c", subcore_axis_name="s")


@functools.partial(
    pl.kernel,
    out_type=jax.ShapeDtypeStruct((4, R * N, QW), jnp.float32),
    mesh=_sc_mesh,
    scratch_types=[
        pltpu.VMEM((GB * CHUNK, QW), jnp.float32),
        pltpu.VMEM((GB * CHUNK, QW), jnp.float32),
        pltpu.VMEM((BPT // 2, CHUNK), jnp.int32),
        pltpu.VMEM((BPT // 2, CHUNK), jnp.int32),
        pltpu.VMEM_SHARED((TBL, QW), jnp.float32),
        pltpu.SemaphoreType.DMA,
        pltpu.SemaphoreType.DMA,
        pltpu.SemaphoreType.DMA,
    ],
    compiler_params=pltpu.CompilerParams(use_tc_tiling_on_sc=False),
)
def _sc_segsum(h4, srcs4, keys, out, rows_v, rows_w, src_all, key_all, table,
               sem0, sem1, sem_s):
    c = lax.axis_index("c")
    s = lax.axis_index("s")

    zero = jnp.zeros((16,), jnp.float32)

    def _zrow(i, carry):
        rows_v[i, pl.ds(0, 16)] = zero
        rows_v[i, pl.ds(16, 16)] = zero
        return carry

    lax.fori_loop(0, GB * CHUNK, _zrow, 0)

    def _zero_table():
        z0 = s * ZPT
        pltpu.sync_copy(rows_v.at[pl.ds(0, 512)], table.at[pl.ds(z0, 512)])
        pltpu.sync_copy(rows_v.at[pl.ds(0, 512)], table.at[pl.ds(z0 + 512, 512)])
        pltpu.sync_copy(rows_v.at[pl.ds(0, 512)], table.at[pl.ds(z0 + 1024, 512)])
        pltpu.sync_copy(rows_v.at[pl.ds(0, ZPT - 1536)],
                        table.at[pl.ds(z0 + 1536, ZPT - 1536)])

    _zero_table()
    plsc.subcore_barrier()

    IDXB = BPT // 2

    for pi in range(2):
        q = 2 * c + pi

        for half in range(2):
            i0 = s * BPT + half * IDXB
            pltpu.sync_copy(srcs4.at[q, pl.ds(i0, IDXB)], src_all)
            pltpu.sync_copy(keys.at[pl.ds(i0, IDXB)], key_all)

            def _drain_body():
                for _ in range(2 * GB):
                    pltpu.make_async_copy(rows_v.at[pl.ds(0, CHUNK)],
                                          table.at[key_all.at[0]],
                                          sem_s).wait()

            def _body(i, carry):
                @pl.when(i != 0)
                def _():
                    _drain_body()

                b0 = i * (2 * GB)
                g0 = [
                    pltpu.async_copy(h4.at[src_all.at[b0 + j]],
                                     rows_v.at[pl.ds(CHUNK * j, CHUNK)], sem0)
                    for j in range(GB)
                ]
                g1 = [
                    pltpu.async_copy(h4.at[src_all.at[b0 + GB + j]],
                                     rows_w.at[pl.ds(CHUNK * j, CHUNK)], sem1)
                    for j in range(GB)
                ]
                for cp in g0:
                    cp.wait()
                for j in range(GB):
                    pltpu.async_copy(rows_v.at[pl.ds(CHUNK * j, CHUNK)],
                                     table.at[key_all.at[b0 + j]], sem_s,
                                     add=True)
                for cp in g1:
                    cp.wait()
                for j in range(GB):
                    pltpu.async_copy(rows_w.at[pl.ds(CHUNK * j, CHUNK)],
                                     table.at[key_all.at[b0 + GB + j]], sem_s,
                                     add=True)
                return carry

            lax.fori_loop(0, IDXB // (2 * GB), _body, 0)
            _drain_body()
        plsc.subcore_barrier()

        o0 = s * CPT
        pltpu.sync_copy(table.at[pl.ds(o0, CPT)], out.at[q, pl.ds(o0, CPT)])

        @pl.when(s == NTILES - 1)
        def _tail():
            t0 = NTILES * CPT
            pltpu.sync_copy(table.at[pl.ds(t0, R * N - t0)],
                            out.at[q, pl.ds(t0, R * N - t0)])

        if pi == 0:
            plsc.subcore_barrier()
            lax.fori_loop(0, GB * CHUNK, _zrow, 0)
            _zero_table()
            plsc.subcore_barrier()


BN = 1000


def _tc_self_body(h_ref, wself_ref, bias_ref, out_ref):
    out_ref[...] = (jnp.dot(h_ref[...], wself_ref[...],
                            preferred_element_type=jnp.float32)
                    + bias_ref[...])


def _tc_self(h, w_self, bias2):
    return pl.pallas_call(
        _tc_self_body,
        grid=(N // BN,),
        in_specs=[
            pl.BlockSpec((BN, D), lambda i: (i, 0)),
            pl.BlockSpec((D, D), lambda i: (0, 0)),
            pl.BlockSpec((1, D), lambda i: (0, 0)),
        ],
        out_specs=pl.BlockSpec((BN, D), lambda i: (i, 0)),
        out_shape=jax.ShapeDtypeStruct((N, D), jnp.float32),
    )(h, w_self, bias2)


def _tc_merge_body(s_ref, hw_ref, wc_ref, bases_ref, out_ref):
    acc = hw_ref[...]
    for r in range(R):
        w_r = (wc_ref[r, 0] * bases_ref[0]
               + wc_ref[r, 1] * bases_ref[1]
               + wc_ref[r, 2] * bases_ref[2])
        s_cat = jnp.concatenate([s_ref[q, r] for q in range(4)], axis=1)
        acc += jnp.dot(s_cat, w_r, preferred_element_type=jnp.float32)
    out_ref[...] = acc


def _tc_merge(s4, hw, w_comp, bases):
    return pl.pallas_call(
        _tc_merge_body,
        grid=(N // BN,),
        in_specs=[
            pl.BlockSpec((4, R, BN, QW), lambda i: (0, 0, i, 0)),
            pl.BlockSpec((BN, D), lambda i: (i, 0)),
            pl.BlockSpec(memory_space=pltpu.SMEM),
            pl.BlockSpec((R, D, D), lambda i: (0, 0, 0)),
        ],
        out_specs=pl.BlockSpec((BN, D), lambda i: (i, 0)),
        out_shape=jax.ShapeDtypeStruct((N, D), jnp.float32),
    )(s4, hw, w_comp, bases)


def kernel(features, edge_index, etypes,
           w_comp0, bases0, w_self0, bias0,
           w_comp1, bases1, w_self1, bias1,
           w_comp2, bases2, w_self2, bias2):
    src = edge_index[0]
    dst = edge_index[1]
    pad = E_PAD - E

    pad_src = jnp.arange(pad, dtype=jnp.int32) % N
    src_full = jnp.concatenate([src, pad_src])
    src4 = 4 * src_full
    srcs4 = jnp.stack([src4, src4 + 1, src4 + 2, src4 + 3])
    srcs4 = srcs4.reshape(4, NBLK, CHUNK)

    keys = etypes * N + dst
    pad_keys = R * N + (jnp.arange(pad, dtype=jnp.int32) % 16)
    keys_b = jnp.concatenate([keys, pad_keys]).reshape(NBLK, CHUNK)

    h = features
    for (wc, ba, ws, bi) in ((w_comp0, bases0, w_self0, bias0),
                             (w_comp1, bases1, w_self1, bias1),
                             (w_comp2, bases2, w_self2, bias2)):
        s_acc = _sc_segsum(h.reshape(4 * N, QW), srcs4, keys_b)
        hw = _tc_self(h, ws, bi.reshape(1, D))
        s4 = s_acc.reshape(4, R, N, QW)
        h = _tc_merge(s4, hw, wc, ba)
    return h
